# baseline (speedup 1.0000x reference)
import jax
import jax.numpy as jnp
from jax import lax
from jax.experimental import pallas as pl
from jax.experimental.pallas import tpu as pltpu

_NUM_EMBEDDINGS = 1024
_EMBEDDING_DIM = 64
_COMMITMENT_COST = 0.25
_BLOCK = 16384


_NCHAIN = 16
_CHAIN = _BLOCK // _NCHAIN


def _vq_half(x, cb, rhs):
    xx = jnp.sum(x * x, axis=1, keepdims=True)
    ee = jnp.sum(cb * cb, axis=1)
    m2xe = lax.dot_general(
        x * -2.0, cb, (((1,), (1,)), ((), ())),
        preferred_element_type=jnp.float32,
        precision=lax.Precision.DEFAULT,
    )
    dist = xx + m2xe + ee[None, :]
    minv = jnp.min(dist, axis=1, keepdims=True)
    maskf = jnp.where(dist == minv, 1.0, 0.0)
    ext = lax.dot_general(
        maskf, rhs, (((1,), (0,)), ((), ())),
        preferred_element_type=jnp.float32,
        precision=lax.Precision.DEFAULT,
    )
    return dist, minv, ext


def _vq_block(x_ref, cb_ref, rhs_ref, q_ref, sse_ref):
    i = pl.program_id(0)
    cb = cb_ref[...]
    rhs = rhs_ref[...]
    chains = [_vq_half(x_ref[pl.ds(c * _CHAIN, _CHAIN)], cb, rhs)
              for c in range(_NCHAIN)]
    for c, (_, _, ext) in enumerate(chains):
        q_ref[pl.ds(c * _CHAIN, _CHAIN)] = ext[:, :_EMBEDDING_DIM]
    cnts = [jnp.max(ext[:, _EMBEDDING_DIM]) for _, _, ext in chains]
    ties = jnp.max(jnp.stack(cnts)) > 1.5

    @pl.when(ties)
    def _fix():
        for c in range(_NCHAIN):
            dist, minv, _ = _vq_half(x_ref[pl.ds(c * _CHAIN, _CHAIN)],
                                     cb, rhs)
            iota = lax.broadcasted_iota(jnp.int32, dist.shape, 1)
            idx = jnp.min(jnp.where(dist == minv, iota, _NUM_EMBEDDINGS),
                          axis=1)
            onehot = jnp.where(iota == idx[:, None], 1.0, 0.0)
            q_ref[pl.ds(c * _CHAIN, _CHAIN)] = lax.dot_general(
                onehot, cb, (((1,), (0,)), ((), ())),
                preferred_element_type=jnp.float32,
                precision=lax.Precision.DEFAULT,
            )

    part = sum(jnp.sum(minv) for _, minv, _ in chains)

    @pl.when(i == 0)
    def _init():
        sse_ref[0, 0] = 0.0

    sse_ref[0, 0] += part


def kernel(inputs, codebook):
    flat = inputs.reshape(-1, _EMBEDDING_DIM)
    n_tokens = flat.shape[0]
    grid = n_tokens // _BLOCK
    rhs = jnp.concatenate(
        [codebook, jnp.ones((_NUM_EMBEDDINGS, 1), jnp.float32)],
        axis=1)
    q, sse = pl.pallas_call(
        _vq_block,
        grid=(grid,),
        in_specs=[
            pl.BlockSpec((_BLOCK, _EMBEDDING_DIM), lambda i: (i, 0)),
            pl.BlockSpec((_NUM_EMBEDDINGS, _EMBEDDING_DIM), lambda i: (0, 0)),
            pl.BlockSpec((_NUM_EMBEDDINGS, _EMBEDDING_DIM + 1),
                         lambda i: (0, 0)),
        ],
        out_specs=[
            pl.BlockSpec((_BLOCK, _EMBEDDING_DIM), lambda i: (i, 0)),
            pl.BlockSpec(memory_space=pltpu.SMEM, block_shape=(1, 1),
                         index_map=lambda i: (0, 0)),
        ],
        out_shape=[
            jax.ShapeDtypeStruct((n_tokens, _EMBEDDING_DIM), jnp.float32),
            jax.ShapeDtypeStruct((1, 1), jnp.float32),
        ],
    )(flat, codebook, rhs)
    loss = sse[0, 0] * (_COMMITMENT_COST / flat.size)
    return (loss, q.reshape(inputs.shape))


# speedup vs baseline: 1.0698x; 1.0698x over previous
import jax
import jax.numpy as jnp
from jax import lax
from jax.experimental import pallas as pl
from jax.experimental.pallas import tpu as pltpu

_NUM_EMBEDDINGS = 1024
_EMBEDDING_DIM = 64
_COMMITMENT_COST = 0.25
_BLOCK = 8192


_NCHAIN = 16
_CHAIN = _BLOCK // _NCHAIN


def _vq_half(x, cb, rhs):
    xx = jnp.sum(x * x, axis=1, keepdims=True)
    ee = jnp.sum(cb * cb, axis=1)
    m2xe = lax.dot_general(
        x * -2.0, cb, (((1,), (1,)), ((), ())),
        preferred_element_type=jnp.float32,
        precision=lax.Precision.DEFAULT,
    )
    dist = xx + m2xe + ee[None, :]
    minv = jnp.min(dist, axis=1, keepdims=True)
    maskf = jnp.where(dist == minv, 1.0, 0.0)
    ext = lax.dot_general(
        maskf, rhs, (((1,), (0,)), ((), ())),
        preferred_element_type=jnp.float32,
        precision=lax.Precision.DEFAULT,
    )
    return dist, minv, ext


def _vq_block(x_ref, cb_ref, rhs_ref, q_ref, sse_ref):
    i = pl.program_id(0)
    cb = cb_ref[...]
    rhs = rhs_ref[...]
    chains = [_vq_half(x_ref[pl.ds(c * _CHAIN, _CHAIN)], cb, rhs)
              for c in range(_NCHAIN)]
    for c, (_, _, ext) in enumerate(chains):
        q_ref[pl.ds(c * _CHAIN, _CHAIN)] = ext[:, :_EMBEDDING_DIM]
    cnts = [jnp.max(ext[:, _EMBEDDING_DIM]) for _, _, ext in chains]
    ties = jnp.max(jnp.stack(cnts)) > 1.5

    @pl.when(ties)
    def _fix():
        for c in range(_NCHAIN):
            dist, minv, _ = _vq_half(x_ref[pl.ds(c * _CHAIN, _CHAIN)],
                                     cb, rhs)
            iota = lax.broadcasted_iota(jnp.int32, dist.shape, 1)
            idx = jnp.min(jnp.where(dist == minv, iota, _NUM_EMBEDDINGS),
                          axis=1)
            onehot = jnp.where(iota == idx[:, None], 1.0, 0.0)
            q_ref[pl.ds(c * _CHAIN, _CHAIN)] = lax.dot_general(
                onehot, cb, (((1,), (0,)), ((), ())),
                preferred_element_type=jnp.float32,
                precision=lax.Precision.DEFAULT,
            )

    part = sum(jnp.sum(minv) for _, minv, _ in chains)

    @pl.when(i == 0)
    def _init():
        sse_ref[0, 0] = 0.0

    sse_ref[0, 0] += part


def kernel(inputs, codebook):
    flat = inputs.reshape(-1, _EMBEDDING_DIM)
    n_tokens = flat.shape[0]
    grid = n_tokens // _BLOCK
    rhs = jnp.concatenate(
        [codebook, jnp.ones((_NUM_EMBEDDINGS, 1), jnp.float32)],
        axis=1)
    q, sse = pl.pallas_call(
        _vq_block,
        grid=(grid,),
        in_specs=[
            pl.BlockSpec((_BLOCK, _EMBEDDING_DIM), lambda i: (i, 0)),
            pl.BlockSpec((_NUM_EMBEDDINGS, _EMBEDDING_DIM), lambda i: (0, 0)),
            pl.BlockSpec((_NUM_EMBEDDINGS, _EMBEDDING_DIM + 1),
                         lambda i: (0, 0)),
        ],
        out_specs=[
            pl.BlockSpec((_BLOCK, _EMBEDDING_DIM), lambda i: (i, 0)),
            pl.BlockSpec(memory_space=pltpu.SMEM, block_shape=(1, 1),
                         index_map=lambda i: (0, 0)),
        ],
        out_shape=[
            jax.ShapeDtypeStruct((n_tokens, _EMBEDDING_DIM), jnp.float32),
            jax.ShapeDtypeStruct((1, 1), jnp.float32),
        ],
    )(flat, codebook, rhs)
    loss = sse[0, 0] * (_COMMITMENT_COST / flat.size)
    return (loss, q.reshape(inputs.shape))
